# baseline (speedup 1.0000x reference)
import functools

import jax
import jax.numpy as jnp
from jax import lax
from jax.experimental import pallas as pl
from jax.experimental.pallas import tpu as pltpu

_VMEM_LIMIT = 60 * 1024 * 1024
_LOG2E = 1.4426950408889634
_NEG = -1e30


def _fused_kernel(x_ref, wq_ref, wk_ref, wv_ref, wo_ref,
                  bq_ref, bk_ref, bv_ref, bo_ref, out_ref,
                  q_sc, k_sc, v_sc, o_sc, *, nh, dh):
    T, E = x_ref.shape

    x = x_ref[...].astype(jnp.bfloat16)
    q_sc[...] = (jnp.dot(x, wq_ref[...], preferred_element_type=jnp.float32)
                 + bq_ref[...]).astype(jnp.bfloat16)
    k_sc[...] = (jnp.dot(x, wk_ref[...], preferred_element_type=jnp.float32)
                 + bk_ref[...]).astype(jnp.bfloat16)
    v_sc[...] = (jnp.dot(x, wv_ref[...], preferred_element_type=jnp.float32)
                 + bv_ref[...]).astype(jnp.bfloat16)

    TQ = T // 2
    Q = TQ // 2
    row_d = lax.broadcasted_iota(jnp.int32, (Q, Q), 0)
    col_d = lax.broadcasted_iota(jnp.int32, (Q, Q), 1)
    dm = jnp.where(col_d <= row_d, jnp.float32(0),
                   jnp.float32(_NEG)).astype(jnp.bfloat16)
    row_r = lax.broadcasted_iota(jnp.int32, (Q, TQ), 0)
    col_r = lax.broadcasted_iota(jnp.int32, (Q, TQ), 1)
    mask_r = jnp.where(col_r <= row_r + Q, jnp.float32(0),
                       jnp.float32(_NEG)).astype(jnp.bfloat16)
    ones_col = jnp.ones((TQ, 1), jnp.bfloat16)

    nt = (((1,), (1,)), ((), ()))

    def att(qx, parts):
        ss = []
        for kx, va, mask in parts:
            s = lax.dot_general(qx, kx, nt, preferred_element_type=jnp.float32
                                ).astype(jnp.bfloat16)
            ss.append(s + mask if mask is not None else s)
        m = None
        for s in ss:
            sm = jnp.max(s, axis=-1, keepdims=True)
            m = sm if m is None else jnp.maximum(m, sm)
        ol = None
        for s, (kx, va, mask) in zip(ss, parts):
            t = jnp.dot(jnp.exp2(s - m), va, preferred_element_type=jnp.float32)
            ol = t if ol is None else ol + t
        return (ol[:, 0:dh] / ol[:, dh:dh + 1]).astype(jnp.bfloat16)

    for h in range(nh):
        sl = pl.ds(h * dh, dh)
        qa = q_sc[0:Q, sl]
        qb = q_sc[Q:TQ, sl]
        qc = q_sc[TQ:TQ + Q, sl]
        qd = q_sc[TQ + Q:T, sl]
        k0 = k_sc[0:TQ, sl]
        k0a = k_sc[0:Q, sl]
        k1 = k_sc[TQ:T, sl]
        k1a = k_sc[TQ:TQ + Q, sl]
        v0a = jnp.concatenate([v_sc[0:TQ, sl], ones_col], axis=1)
        v1a = jnp.concatenate([v_sc[TQ:T, sl], ones_col], axis=1)
        v0h = v0a[0:Q]
        v1h = v1a[0:Q]

        o_sc[0:Q, sl] = att(qa, [(k0a, v0h, dm)])
        o_sc[Q:TQ, sl] = att(qb, [(k0, v0a, mask_r)])
        o_sc[TQ:TQ + Q, sl] = att(qc, [(k0, v0a, None), (k1a, v1h, dm)])
        o_sc[TQ + Q:T, sl] = att(qd, [(k0, v0a, None), (k1, v1a, mask_r)])

    out = jnp.dot(o_sc[...], wo_ref[...], preferred_element_type=jnp.float32)
    out_ref[...] = out + bo_ref[...]


def kernel(hidden_states, q_w, q_b, k_w, k_b, v_w, v_b, o_w, o_b):
    B, T, E = hidden_states.shape
    H = 16
    Dh = E // H
    scale = float(Dh) ** -0.5 * _LOG2E

    wq = (q_w * scale).astype(jnp.bfloat16)
    bq = (q_b * scale).reshape(1, E)

    x_spec = pl.BlockSpec((None, T, E), lambda b: (b, 0, 0))
    w_spec = pl.BlockSpec((E, E), lambda b: (0, 0))
    b_spec = pl.BlockSpec((1, E), lambda b: (0, 0))

    out = pl.pallas_call(
        functools.partial(_fused_kernel, nh=H, dh=Dh),
        out_shape=jax.ShapeDtypeStruct((B, T, E), jnp.float32),
        grid=(B,),
        in_specs=[x_spec, w_spec, w_spec, w_spec, w_spec,
                  b_spec, b_spec, b_spec, b_spec],
        out_specs=x_spec,
        scratch_shapes=[
            pltpu.VMEM((T, E), jnp.bfloat16),
            pltpu.VMEM((T, E), jnp.bfloat16),
            pltpu.VMEM((T, E), jnp.bfloat16),
            pltpu.VMEM((T, E), jnp.bfloat16),
        ],
        compiler_params=pltpu.CompilerParams(
            dimension_semantics=("parallel",),
            vmem_limit_bytes=_VMEM_LIMIT),
    )(hidden_states, wq,
      k_w.astype(jnp.bfloat16), v_w.astype(jnp.bfloat16),
      o_w.astype(jnp.bfloat16),
      bq, k_b.reshape(1, E), v_b.reshape(1, E), o_b.reshape(1, E))
    return out


# speedup vs baseline: 1.3357x; 1.3357x over previous
import functools

import jax
import jax.numpy as jnp
from jax import lax
from jax.experimental import pallas as pl
from jax.experimental.pallas import tpu as pltpu

_VMEM_LIMIT = 60 * 1024 * 1024
_LOG2E = 1.4426950408889634
_NEG = -1e30


def _fused_kernel(x_ref, wq_ref, wk_ref, wv_ref, wo_ref,
                  bq_ref, bk_ref, bv_ref, bo_ref, out_ref,
                  q_sc, k_sc, v_sc, o_sc, *, nh, dh):
    T, E = x_ref.shape

    x = x_ref[...].astype(jnp.bfloat16)
    q_sc[...] = (jnp.dot(x, wq_ref[...], preferred_element_type=jnp.float32)
                 + bq_ref[...]).astype(jnp.bfloat16)
    k_sc[...] = (jnp.dot(x, wk_ref[...], preferred_element_type=jnp.float32)
                 + bk_ref[...]).astype(jnp.bfloat16)
    v_sc[...] = (jnp.dot(x, wv_ref[...], preferred_element_type=jnp.float32)
                 + bv_ref[...]).astype(jnp.bfloat16)

    TQ = T // 2
    row_d = lax.broadcasted_iota(jnp.int32, (TQ, TQ), 0)
    col_d = lax.broadcasted_iota(jnp.int32, (TQ, TQ), 1)
    diag_mask = jnp.where(col_d <= row_d, jnp.float32(0),
                          jnp.float32(_NEG)).astype(jnp.bfloat16)
    ones_col = jnp.ones((TQ, 1), jnp.bfloat16)

    nt = (((1,), (1,)), ((), ()))

    for h in range(nh):
        sl = pl.ds(h * dh, dh)
        q0 = q_sc[0:TQ, sl]
        q1 = q_sc[TQ:T, sl]
        k0 = k_sc[0:TQ, sl]
        k1 = k_sc[TQ:T, sl]
        v0a = jnp.concatenate([v_sc[0:TQ, sl], ones_col], axis=1)
        v1a = jnp.concatenate([v_sc[TQ:T, sl], ones_col], axis=1)

        s00 = lax.dot_general(q0, k0, nt, preferred_element_type=jnp.float32
                              ).astype(jnp.bfloat16) + diag_mask
        m0 = jnp.max(s00, axis=-1, keepdims=True)
        p00 = jnp.exp2(s00 - m0)
        ol0 = jnp.dot(p00, v0a, preferred_element_type=jnp.float32)
        o_sc[0:TQ, sl] = (ol0[:, 0:dh] / ol0[:, dh:dh + 1]).astype(jnp.bfloat16)

        s10 = lax.dot_general(q1, k0, nt, preferred_element_type=jnp.float32
                              ).astype(jnp.bfloat16)
        s11 = lax.dot_general(q1, k1, nt, preferred_element_type=jnp.float32
                              ).astype(jnp.bfloat16) + diag_mask
        m1 = jnp.maximum(jnp.max(s10, axis=-1, keepdims=True),
                         jnp.max(s11, axis=-1, keepdims=True))
        p10 = jnp.exp2(s10 - m1)
        p11 = jnp.exp2(s11 - m1)
        ol1 = (jnp.dot(p10, v0a, preferred_element_type=jnp.float32)
               + jnp.dot(p11, v1a, preferred_element_type=jnp.float32))
        o_sc[TQ:T, sl] = (ol1[:, 0:dh] / ol1[:, dh:dh + 1]).astype(jnp.bfloat16)

    out = jnp.dot(o_sc[...], wo_ref[...], preferred_element_type=jnp.float32)
    out_ref[...] = out + bo_ref[...]


def kernel(hidden_states, q_w, q_b, k_w, k_b, v_w, v_b, o_w, o_b):
    B, T, E = hidden_states.shape
    H = 16
    Dh = E // H
    scale = float(Dh) ** -0.5 * _LOG2E

    wq = (q_w * scale).astype(jnp.bfloat16)
    bq = (q_b * scale).reshape(1, E)

    x_spec = pl.BlockSpec((None, T, E), lambda b: (b, 0, 0))
    w_spec = pl.BlockSpec((E, E), lambda b: (0, 0))
    b_spec = pl.BlockSpec((1, E), lambda b: (0, 0))

    out = pl.pallas_call(
        functools.partial(_fused_kernel, nh=H, dh=Dh),
        out_shape=jax.ShapeDtypeStruct((B, T, E), jnp.float32),
        grid=(B,),
        in_specs=[x_spec, w_spec, w_spec, w_spec, w_spec,
                  b_spec, b_spec, b_spec, b_spec],
        out_specs=x_spec,
        scratch_shapes=[
            pltpu.VMEM((T, E), jnp.bfloat16),
            pltpu.VMEM((T, E), jnp.bfloat16),
            pltpu.VMEM((T, E), jnp.bfloat16),
            pltpu.VMEM((T, E), jnp.bfloat16),
        ],
        compiler_params=pltpu.CompilerParams(
            dimension_semantics=("parallel",),
            vmem_limit_bytes=_VMEM_LIMIT),
    )(hidden_states, wq,
      k_w.astype(jnp.bfloat16), v_w.astype(jnp.bfloat16),
      o_w.astype(jnp.bfloat16),
      bq, k_b.reshape(1, E), v_b.reshape(1, E), o_b.reshape(1, E))
    return out
